# baseline (speedup 1.0000x reference)
import jax
import jax.numpy as jnp
from jax import lax
from jax.experimental import pallas as pl
from jax.experimental.pallas import tpu as pltpu

_K_TOP = 256
_CELLS_PER_ROW = 8


def _cellcnn_body(xw_ref, bd_ref, b1_ref, w2_ref, b2_ref, out_ref, hall_ref):
    B, nr, nl = hall_ref.shape
    nf = nl // _CELLS_PER_ROW
    step = pl.program_id(0)

    @pl.when(step < B)
    def _matmul():
        x = xw_ref[0]
        h = jnp.dot(x, bd_ref[...], preferred_element_type=jnp.float32)
        hall_ref[step] = jnp.maximum(h + b1_ref[...], 0.0)

    @pl.when(step == B)
    def _search():
        li = lax.broadcasted_iota(jnp.int32, (nl, nl), 0)
        mi = lax.broadcasted_iota(jnp.int32, (nl, nl), 1)
        foldm = jnp.where((li % nf) == (mi % nf), 1.0, 0.0).astype(jnp.float32)

        CH = 8
        U = 16

        def count_ge_b(b, t_b):
            def body(i, accs):
                base = i * (CH * U)
                return tuple(
                    accs[j]
                    + jnp.where(
                        hall_ref[b, pl.ds(base + j * CH, CH), :] >= t_b,
                        1.0,
                        0.0,
                    )
                    for j in range(U)
                )
            accs = lax.fori_loop(
                0,
                nr // (CH * U),
                body,
                tuple(jnp.zeros((CH, nl), jnp.float32) for _ in range(U)),
            )
            tot = accs[0]
            for j in range(1, U):
                tot = tot + accs[j]
            return jnp.sum(tot, axis=0, keepdims=True)

        def count_ge(t_bits):
            t = lax.bitcast_convert_type(t_bits, jnp.float32)
            cnt = jnp.concatenate(
                [count_ge_b(b, t[b : b + 1]) for b in range(B)], axis=0
            )
            return jnp.dot(cnt, foldm, preferred_element_type=jnp.float32)

        def bs_body(_, carry):
            lo, hi = carry
            mid = lo + lax.div(hi - lo, 2)
            pred = count_ge(mid) >= float(_K_TOP)
            return jnp.where(pred, mid, lo), jnp.where(pred, hi, mid)

        lo0 = jnp.zeros((B, nl), jnp.int32)
        hi0 = jnp.full((B, nl), jnp.int32(2**31 - 1))
        lo, hi = lax.fori_loop(0, 31, bs_body, (lo0, hi0))

        t_lo = lax.bitcast_convert_type(lo, jnp.float32)
        t_hi = lax.bitcast_convert_type(hi, jnp.float32)

        def sum_cnt_gt_b(b, t_b):
            def body(i, carry):
                saccs, caccs = carry
                base = i * (CH * U)
                s_new, c_new = [], []
                for j in range(U):
                    blk = hall_ref[b, pl.ds(base + j * CH, CH), :]
                    m = blk >= t_b
                    s_new.append(saccs[j] + jnp.where(m, blk, 0.0))
                    c_new.append(caccs[j] + jnp.where(m, 1.0, 0.0))
                return tuple(s_new), tuple(c_new)
            z16 = tuple(jnp.zeros((CH, nl), jnp.float32) for _ in range(U))
            saccs, caccs = lax.fori_loop(
                0, nr // (CH * U), body, (z16, z16)
            )
            stot, ctot = saccs[0], caccs[0]
            for j in range(1, U):
                stot = stot + saccs[j]
                ctot = ctot + caccs[j]
            return (
                jnp.sum(stot, axis=0, keepdims=True),
                jnp.sum(ctot, axis=0, keepdims=True),
            )

        res = [sum_cnt_gt_b(b, t_hi[b : b + 1]) for b in range(B)]
        sums = jnp.concatenate([r[0] for r in res], axis=0)
        cgt = jnp.concatenate([r[1] for r in res], axis=0)
        sumsf = jnp.dot(sums, foldm, preferred_element_type=jnp.float32)
        cgtf = jnp.dot(cgt, foldm, preferred_element_type=jnp.float32)
        sum_top = sumsf + (float(_K_TOP) - cgtf) * t_lo
        pooled = sum_top[:, :nf] * (1.0 / _K_TOP)

        z = jnp.sum(pooled * w2_ref[...], axis=1, keepdims=True) + b2_ref[...]
        out_ref[...] = (1.0 / (1.0 + jnp.exp(-z))).reshape(B, 1, 1)


def _build_call(B, NR, D, F):
    C = _CELLS_PER_ROW
    return pl.pallas_call(
        _cellcnn_body,
        grid=(B + 1,),
        in_specs=[
            pl.BlockSpec((1, NR, C * D), lambda b: (jnp.minimum(b, B - 1), 0, 0)),
            pl.BlockSpec((C * D, C * F), lambda b: (0, 0)),
            pl.BlockSpec((1, C * F), lambda b: (0, 0)),
            pl.BlockSpec((1, F), lambda b: (0, 0)),
            pl.BlockSpec((1, 1), lambda b: (0, 0)),
        ],
        out_specs=pl.BlockSpec((B, 1, 1), lambda b: (0, 0, 0)),
        out_shape=jax.ShapeDtypeStruct((B, 1, 1), jnp.float32),
        scratch_shapes=[pltpu.VMEM((B, NR, C * F), jnp.float32)],
    )


def kernel(inputs, W1, b1, W2, b2):
    B, N, D = inputs.shape
    F = W1.shape[1]
    C = _CELLS_PER_ROW
    NR = N // C
    xw = inputs.reshape(B, NR, C * D)
    eye = jnp.eye(C, dtype=W1.dtype)
    bd = jnp.einsum("ce,df->cdef", eye, W1).reshape(C * D, C * F)
    b1t = jnp.tile(b1, C).reshape(1, C * F)
    w2t = W2.reshape(1, F)
    b2r = b2.reshape(1, 1)
    out = _build_call(B, NR, D, F)(xw, bd, b1t, w2t, b2r)
    return out.reshape(B, 1)


# speedup vs baseline: 1.0190x; 1.0190x over previous
import jax
import jax.numpy as jnp
from jax import lax
from jax.experimental import pallas as pl
from jax.experimental.pallas import tpu as pltpu

_K_TOP = 256
_CELLS_PER_ROW = 8


def _cellcnn_body(xw_ref, bd_ref, b1_ref, w2_ref, b2_ref, out_ref, hall_ref):
    B, nr, nl = hall_ref.shape
    nf = nl // _CELLS_PER_ROW
    step = pl.program_id(0)

    @pl.when(step < B)
    def _matmul():
        x = xw_ref[0]
        h = jnp.dot(x, bd_ref[...], preferred_element_type=jnp.float32)
        hall_ref[step] = jnp.maximum(h + b1_ref[...], 0.0)

    @pl.when(step == B)
    def _search():
        li = lax.broadcasted_iota(jnp.int32, (nl, nl), 0)
        mi = lax.broadcasted_iota(jnp.int32, (nl, nl), 1)
        foldm = jnp.where((li % nf) == (mi % nf), 1.0, 0.0).astype(jnp.float32)

        CR = 128

        def count_ge_b(b, t_b):
            def body(i, acc):
                blk = hall_ref[b, pl.ds(i * CR, CR), :]
                return acc + jnp.where(blk >= t_b, 1.0, 0.0)
            acc = lax.fori_loop(
                0, nr // CR, body, jnp.zeros((CR, nl), jnp.float32)
            )
            return jnp.sum(acc, axis=0, keepdims=True)

        def count_ge(t_bits):
            t = lax.bitcast_convert_type(t_bits, jnp.float32)
            cnt = jnp.concatenate(
                [count_ge_b(b, t[b : b + 1]) for b in range(B)], axis=0
            )
            return jnp.dot(cnt, foldm, preferred_element_type=jnp.float32)

        def bs_body(_, carry):
            lo, hi = carry
            mid = lo + lax.div(hi - lo, 2)
            pred = count_ge(mid) >= float(_K_TOP)
            return jnp.where(pred, mid, lo), jnp.where(pred, hi, mid)

        lo0 = jnp.zeros((B, nl), jnp.int32)
        hi0 = jnp.full((B, nl), jnp.int32(2**31 - 1))
        lo, hi = lax.fori_loop(0, 31, bs_body, (lo0, hi0))

        t_lo = lax.bitcast_convert_type(lo, jnp.float32)
        t_hi = lax.bitcast_convert_type(hi, jnp.float32)

        def sum_cnt_gt_b(b, t_b):
            def body(i, carry):
                sacc, cacc = carry
                blk = hall_ref[b, pl.ds(i * CR, CR), :]
                m = blk >= t_b
                return (
                    sacc + jnp.where(m, blk, 0.0),
                    cacc + jnp.where(m, 1.0, 0.0),
                )
            z = jnp.zeros((CR, nl), jnp.float32)
            sacc, cacc = lax.fori_loop(0, nr // CR, body, (z, z))
            return (
                jnp.sum(sacc, axis=0, keepdims=True),
                jnp.sum(cacc, axis=0, keepdims=True),
            )

        res = [sum_cnt_gt_b(b, t_hi[b : b + 1]) for b in range(B)]
        sums = jnp.concatenate([r[0] for r in res], axis=0)
        cgt = jnp.concatenate([r[1] for r in res], axis=0)
        sumsf = jnp.dot(sums, foldm, preferred_element_type=jnp.float32)
        cgtf = jnp.dot(cgt, foldm, preferred_element_type=jnp.float32)
        sum_top = sumsf + (float(_K_TOP) - cgtf) * t_lo
        pooled = sum_top[:, :nf] * (1.0 / _K_TOP)

        z = jnp.sum(pooled * w2_ref[...], axis=1, keepdims=True) + b2_ref[...]
        out_ref[...] = (1.0 / (1.0 + jnp.exp(-z))).reshape(B, 1, 1)


def _build_call(B, NR, D, F):
    C = _CELLS_PER_ROW
    return pl.pallas_call(
        _cellcnn_body,
        grid=(B + 1,),
        in_specs=[
            pl.BlockSpec((1, NR, C * D), lambda b: (jnp.minimum(b, B - 1), 0, 0)),
            pl.BlockSpec((C * D, C * F), lambda b: (0, 0)),
            pl.BlockSpec((1, C * F), lambda b: (0, 0)),
            pl.BlockSpec((1, F), lambda b: (0, 0)),
            pl.BlockSpec((1, 1), lambda b: (0, 0)),
        ],
        out_specs=pl.BlockSpec((B, 1, 1), lambda b: (0, 0, 0)),
        out_shape=jax.ShapeDtypeStruct((B, 1, 1), jnp.float32),
        scratch_shapes=[pltpu.VMEM((B, NR, C * F), jnp.float32)],
    )


def kernel(inputs, W1, b1, W2, b2):
    B, N, D = inputs.shape
    F = W1.shape[1]
    C = _CELLS_PER_ROW
    NR = N // C
    xw = inputs.reshape(B, NR, C * D)
    eye = jnp.eye(C, dtype=W1.dtype)
    bd = jnp.einsum("ce,df->cdef", eye, W1).reshape(C * D, C * F)
    b1t = jnp.tile(b1, C).reshape(1, C * F)
    w2t = W2.reshape(1, F)
    b2r = b2.reshape(1, 1)
    out = _build_call(B, NR, D, F)(xw, bd, b1t, w2t, b2r)
    return out.reshape(B, 1)
